# Initial kernel scaffold; baseline (speedup 1.0000x reference)
import functools

import jax
import jax.numpy as jnp
from jax import lax
from jax.experimental import pallas as pl
from jax.experimental.pallas import tpu as pltpu
from jax.experimental.pallas import tpu_sc as plsc

VOCAB = 100000
DIM = 64
BATCH = 1024
NB_WORDS = 200
B = BATCH * NB_WORDS

_info = plsc.get_sparse_core_info()
NC, NS = _info.num_cores, _info.num_subcores
NW = NC * NS
B_PER_W = B // NW
CHUNK = 800
N_CHUNKS = B_PER_W // CHUNK


def _make_kernel():
    mesh = plsc.VectorSubcoreMesh(core_axis_name="c", subcore_axis_name="s")

    @functools.partial(
        pl.kernel,
        mesh=mesh,
        out_type=jax.ShapeDtypeStruct((B, 2 * DIM), jnp.float32),
        scratch_types=[
            pltpu.VMEM((CHUNK,), jnp.int32),
            pltpu.VMEM((CHUNK, DIM), jnp.float32),
            pltpu.VMEM((CHUNK, DIM), jnp.float32),
            pltpu.SemaphoreType.DMA,
            pltpu.SemaphoreType.DMA,
        ],
    )
    def emb_kernel(x_hbm, glove_hbm, rand_hbm, out_hbm, idx_v, g_v, r_v,
                   sem_g, sem_r):
        wid = lax.axis_index("s") * NC + lax.axis_index("c")
        base = wid * B_PER_W

        def body(i, carry):
            off = base + i * CHUNK
            pltpu.sync_copy(x_hbm.at[pl.ds(off, CHUNK)], idx_v)
            cg = pltpu.async_copy(glove_hbm.at[idx_v], g_v, sem_g)
            cr = pltpu.async_copy(rand_hbm.at[idx_v], r_v, sem_r)
            cg.wait()
            cr.wait()
            pltpu.sync_copy(g_v, out_hbm.at[pl.ds(off, CHUNK), pl.ds(0, DIM)])
            pltpu.sync_copy(r_v, out_hbm.at[pl.ds(off, CHUNK), pl.ds(DIM, DIM)])
            return carry

        lax.fori_loop(0, N_CHUNKS, body, 0)

    return emb_kernel


_emb = _make_kernel()


@jax.jit
def kernel(x, glove_table, rand_table):
    x_flat = x.reshape(B).astype(jnp.int32)
    out = _emb(x_flat, glove_table, rand_table)
    return out.reshape(BATCH, NB_WORDS, 2 * DIM)


# speedup vs baseline: 7.8097x; 7.8097x over previous
import functools

import jax
import jax.numpy as jnp
from jax import lax
from jax.experimental import pallas as pl
from jax.experimental.pallas import tpu as pltpu
from jax.experimental.pallas import tpu_sc as plsc

VOCAB = 100000
DIM = 64
BATCH = 1024
NB_WORDS = 200
B = BATCH * NB_WORDS

_info = plsc.get_sparse_core_info()
NC, NS = _info.num_cores, _info.num_subcores
NW = NC * NS
B_PER_W = B // NW
CHUNK = 800
N_CHUNKS = B_PER_W // CHUNK


def _make_kernel():
    mesh = plsc.VectorSubcoreMesh(core_axis_name="c", subcore_axis_name="s")

    @functools.partial(
        pl.kernel,
        mesh=mesh,
        out_type=jax.ShapeDtypeStruct((B, 2 * DIM), jnp.float32),
        compiler_params=pltpu.CompilerParams(use_tc_tiling_on_sc=False),
        scratch_types=[
            pltpu.VMEM((CHUNK,), jnp.int32),
            pltpu.VMEM((CHUNK, DIM), jnp.float32),
            pltpu.VMEM((CHUNK, DIM), jnp.float32),
            pltpu.SemaphoreType.DMA,
            pltpu.SemaphoreType.DMA,
        ],
    )
    def emb_kernel(x_hbm, glove_hbm, rand_hbm, out_hbm, idx_v, g_v, r_v,
                   sem_g, sem_r):
        wid = lax.axis_index("s") * NC + lax.axis_index("c")
        base = wid * B_PER_W

        def body(i, carry):
            off = base + i * CHUNK
            pltpu.sync_copy(x_hbm.at[pl.ds(off, CHUNK)], idx_v)
            cg = pltpu.async_copy(glove_hbm.at[idx_v], g_v, sem_g)
            cr = pltpu.async_copy(rand_hbm.at[idx_v], r_v, sem_r)
            cg.wait()
            cr.wait()
            pltpu.sync_copy(g_v, out_hbm.at[pl.ds(off, CHUNK), pl.ds(0, DIM)])
            pltpu.sync_copy(r_v, out_hbm.at[pl.ds(off, CHUNK), pl.ds(DIM, DIM)])
            return carry

        lax.fori_loop(0, N_CHUNKS, body, 0)

    return emb_kernel


_emb = _make_kernel()


@jax.jit
def kernel(x, glove_table, rand_table):
    x_flat = x.reshape(B).astype(jnp.int32)
    out = _emb(x_flat, glove_table, rand_table)
    return out.reshape(BATCH, NB_WORDS, 2 * DIM)


# speedup vs baseline: 7.9393x; 1.0166x over previous
import functools

import jax
import jax.numpy as jnp
from jax import lax
from jax.experimental import pallas as pl
from jax.experimental.pallas import tpu as pltpu
from jax.experimental.pallas import tpu_sc as plsc

VOCAB = 100000
DIM = 64
BATCH = 1024
NB_WORDS = 200
B = BATCH * NB_WORDS

_info = plsc.get_sparse_core_info()
NC, NS = _info.num_cores, _info.num_subcores
NW = NC * NS
B_PER_W = B // NW
CHUNK = 320
N_CHUNKS = B_PER_W // CHUNK
NBUF = 3


def _make_kernel():
    mesh = plsc.VectorSubcoreMesh(core_axis_name="c", subcore_axis_name="s")

    row_buf = pltpu.VMEM((CHUNK, DIM), jnp.float32)

    @functools.partial(
        pl.kernel,
        mesh=mesh,
        out_type=jax.ShapeDtypeStruct((B, 2 * DIM), jnp.float32),
        compiler_params=pltpu.CompilerParams(use_tc_tiling_on_sc=False),
        scratch_types=(
            [pltpu.VMEM((B_PER_W,), jnp.int32)]
            + [row_buf] * NBUF
            + [row_buf] * NBUF
            + [pltpu.SemaphoreType.DMA] * NBUF
            + [pltpu.SemaphoreType.DMA] * NBUF
        ),
    )
    def emb_kernel(x_hbm, glove_hbm, rand_hbm, out_hbm, idx_all,
                   g0, g1, g2, r0, r1, r2, sg0, sg1, sg2, so0, so1, so2):
        g_bufs = [g0, g1, g2]
        r_bufs = [r0, r1, r2]
        g_sems = [sg0, sg1, sg2]
        o_sems = [so0, so1, so2]

        wid = lax.axis_index("s") * NC + lax.axis_index("c")
        base = wid * B_PER_W
        pltpu.sync_copy(x_hbm.at[pl.ds(base, B_PER_W)], idx_all)

        gather_descs = [None] * N_CHUNKS
        out_descs = [None] * N_CHUNKS

        def start_gathers(i):
            p = i % NBUF
            idx = idx_all.at[pl.ds(i * CHUNK, CHUNK)]
            cg = pltpu.async_copy(glove_hbm.at[idx], g_bufs[p], g_sems[p])
            cr = pltpu.async_copy(rand_hbm.at[idx], r_bufs[p], g_sems[p])
            gather_descs[i] = (cg, cr)

        def start_out(i):
            p = i % NBUF
            off = base + i * CHUNK
            rows = out_hbm.at[pl.ds(off, CHUNK), pl.ds(0, DIM)]
            co_g = pltpu.async_copy(g_bufs[p], rows, o_sems[p])
            rows = out_hbm.at[pl.ds(off, CHUNK), pl.ds(DIM, DIM)]
            co_r = pltpu.async_copy(r_bufs[p], rows, o_sems[p])
            out_descs[i] = (co_g, co_r)

        for i in range(N_CHUNKS):
            if i >= NBUF:
                for d in out_descs[i - NBUF]:
                    d.wait()
            start_gathers(i)
            if i >= 1:
                for d in gather_descs[i - 1]:
                    d.wait()
                start_out(i - 1)

        for d in gather_descs[N_CHUNKS - 1]:
            d.wait()
        start_out(N_CHUNKS - 1)
        for i in range(N_CHUNKS - NBUF, N_CHUNKS):
            for d in out_descs[i]:
                d.wait()

    return emb_kernel


_emb = _make_kernel()


@jax.jit
def kernel(x, glove_table, rand_table):
    x_flat = x.reshape(B).astype(jnp.int32)
    out = _emb(x_flat, glove_table, rand_table)
    return out.reshape(BATCH, NB_WORDS, 2 * DIM)


# speedup vs baseline: 8.7045x; 1.0964x over previous
import functools

import jax
import jax.numpy as jnp
from jax import lax
from jax.experimental import pallas as pl
from jax.experimental.pallas import tpu as pltpu
from jax.experimental.pallas import tpu_sc as plsc

VOCAB = 100000
DIM = 64
BATCH = 1024
NB_WORDS = 200
B = BATCH * NB_WORDS

_info = plsc.get_sparse_core_info()
NC, NS = _info.num_cores, _info.num_subcores
NW = NC * NS
B_PER_W = B // NW
CHUNK = 320
N_CHUNKS = B_PER_W // CHUNK
NBUF = 3


def _make_kernel():
    mesh = plsc.VectorSubcoreMesh(core_axis_name="c", subcore_axis_name="s")

    comb_buf = pltpu.VMEM((CHUNK, 2 * DIM), jnp.float32)

    @functools.partial(
        pl.kernel,
        mesh=mesh,
        out_type=jax.ShapeDtypeStruct((B, 2 * DIM), jnp.float32),
        compiler_params=pltpu.CompilerParams(use_tc_tiling_on_sc=False),
        scratch_types=(
            [pltpu.VMEM((B_PER_W,), jnp.int32)]
            + [comb_buf] * NBUF
            + [pltpu.SemaphoreType.DMA] * NBUF
            + [pltpu.SemaphoreType.DMA] * NBUF
        ),
    )
    def emb_kernel(x_hbm, table_hbm, out_hbm, idx_all,
                   c0, c1, c2, sg0, sg1, sg2, so0, so1, so2):
        bufs = [c0, c1, c2]
        g_sems = [sg0, sg1, sg2]
        o_sems = [so0, so1, so2]

        wid = lax.axis_index("s") * NC + lax.axis_index("c")
        base = wid * B_PER_W
        pltpu.sync_copy(x_hbm.at[pl.ds(base, B_PER_W)], idx_all)

        gather_descs = [None] * N_CHUNKS
        out_descs = [None] * N_CHUNKS

        def start_gather(i):
            p = i % NBUF
            idx = idx_all.at[pl.ds(i * CHUNK, CHUNK)]
            cg = pltpu.async_copy(table_hbm.at[idx], bufs[p], g_sems[p])
            gather_descs[i] = cg

        def start_out(i):
            p = i % NBUF
            off = base + i * CHUNK
            co = pltpu.async_copy(bufs[p], out_hbm.at[pl.ds(off, CHUNK)],
                                  o_sems[p])
            out_descs[i] = co

        for i in range(N_CHUNKS):
            if i >= NBUF:
                out_descs[i - NBUF].wait()
            start_gather(i)
            if i >= 1:
                gather_descs[i - 1].wait()
                start_out(i - 1)

        gather_descs[N_CHUNKS - 1].wait()
        start_out(N_CHUNKS - 1)
        for i in range(N_CHUNKS - NBUF, N_CHUNKS):
            out_descs[i].wait()

    return emb_kernel


_emb = _make_kernel()


@jax.jit
def kernel(x, glove_table, rand_table):
    comb = jnp.concatenate([glove_table, rand_table], axis=1)
    x_flat = x.reshape(B).astype(jnp.int32)
    out = _emb(x_flat, comb)
    return out.reshape(BATCH, NB_WORDS, 2 * DIM)


# speedup vs baseline: 8.8945x; 1.0218x over previous
import functools

import jax
import jax.numpy as jnp
from jax import lax
from jax.experimental import pallas as pl
from jax.experimental.pallas import tpu as pltpu
from jax.experimental.pallas import tpu_sc as plsc

VOCAB = 100000
DIM = 64
BATCH = 1024
NB_WORDS = 200
B = BATCH * NB_WORDS

_info = plsc.get_sparse_core_info()
NC, NS = _info.num_cores, _info.num_subcores
NW = NC * NS
B_PER_W = B // NW
SUB = 128
NRING = 5
N_SUB = B_PER_W // SUB
N_GROUP = N_SUB // NRING


def _make_kernel():
    mesh = plsc.VectorSubcoreMesh(core_axis_name="c", subcore_axis_name="s")

    ring_buf = pltpu.VMEM((SUB, 2 * DIM), jnp.float32)

    @functools.partial(
        pl.kernel,
        mesh=mesh,
        out_type=jax.ShapeDtypeStruct((B, 2 * DIM), jnp.float32),
        compiler_params=pltpu.CompilerParams(use_tc_tiling_on_sc=False),
        scratch_types=(
            [pltpu.VMEM((B_PER_W,), jnp.int32)]
            + [ring_buf] * NRING
            + [pltpu.SemaphoreType.DMA] * NRING
            + [pltpu.SemaphoreType.DMA] * NRING
        ),
    )
    def emb_kernel(x_hbm, table_hbm, out_hbm, idx_all,
                   b0, b1, b2, b3, b4,
                   sg0, sg1, sg2, sg3, sg4,
                   so0, so1, so2, so3, so4):
        bufs = [b0, b1, b2, b3, b4]
        g_sems = [sg0, sg1, sg2, sg3, sg4]
        o_sems = [so0, so1, so2, so3, so4]

        wid = lax.axis_index("s") * NC + lax.axis_index("c")
        base = wid * B_PER_W
        pltpu.sync_copy(x_hbm.at[pl.ds(base, B_PER_W)], idx_all)

        def gather_desc(i, p):
            idx = idx_all.at[pl.ds(i * SUB, SUB)]
            return pltpu.make_async_copy(table_hbm.at[idx], bufs[p],
                                         g_sems[p])

        def out_desc(i, p):
            dst = out_hbm.at[pl.ds(base + i * SUB, SUB)]
            return pltpu.make_async_copy(bufs[p], dst, o_sems[p])


        gather_desc(0, 0).start()
        for b in range(NRING):
            if b + 1 >= NRING:
                out_desc(b - NRING + 1, (b + 1) % NRING).wait()
            gather_desc(b + 1, (b + 1) % NRING).start()
            gather_desc(b, b).wait()
            out_desc(b, b).start()

        @pl.loop(1, N_GROUP - 1)
        def _group(g):
            i0 = g * NRING
            for b in range(NRING):
                i = i0 + b
                q = (b + 1) % NRING
                out_desc(i - NRING + 1, q).wait()
                gather_desc(i + 1, q).start()
                gather_desc(i, b).wait()
                out_desc(i, b).start()

        i0 = (N_GROUP - 1) * NRING
        for b in range(NRING):
            i = i0 + b
            q = (b + 1) % NRING
            out_desc(i - NRING + 1, q).wait()
            if b + 1 < NRING:
                gather_desc(i + 1, q).start()
            gather_desc(i, b).wait()
            out_desc(i, b).start()
        for b in range(1, NRING):
            out_desc(i0 + b, b).wait()

    return emb_kernel


_emb = _make_kernel()


@jax.jit
def kernel(x, glove_table, rand_table):
    comb = jnp.concatenate([glove_table, rand_table], axis=1)
    x_flat = x.reshape(B).astype(jnp.int32)
    out = _emb(x_flat, comb)
    return out.reshape(BATCH, NB_WORDS, 2 * DIM)


# speedup vs baseline: 10.2834x; 1.1562x over previous
import functools

import jax
import jax.numpy as jnp
from jax import lax
from jax.experimental import pallas as pl
from jax.experimental.pallas import tpu as pltpu
from jax.experimental.pallas import tpu_sc as plsc

VOCAB = 100000
DIM = 64
BATCH = 1024
NB_WORDS = 200
B = BATCH * NB_WORDS

_info = plsc.get_sparse_core_info()
NC, NS = _info.num_cores, _info.num_subcores
NW = NC * NS
B_PER_W = B // NW
SUB = 128
NRING = 5
N_SUB = B_PER_W // SUB
N_GROUP = N_SUB // NRING


def _make_kernel():
    mesh = plsc.VectorSubcoreMesh(core_axis_name="c", subcore_axis_name="s")

    ring_buf = pltpu.VMEM((SUB, 2 * DIM), jnp.float32)

    @functools.partial(
        pl.kernel,
        mesh=mesh,
        out_type=jax.ShapeDtypeStruct((B, 2 * DIM), jnp.float32),
        compiler_params=pltpu.CompilerParams(use_tc_tiling_on_sc=False),
        scratch_types=(
            [pltpu.VMEM((B_PER_W,), jnp.int32)]
            + [ring_buf] * NRING
            + [pltpu.SemaphoreType.DMA] * NRING
            + [pltpu.SemaphoreType.DMA] * NRING
        ),
    )
    def emb_kernel(x_hbm, table_hbm, out_hbm, idx_all,
                   b0, b1, b2, b3, b4,
                   sg0, sg1, sg2, sg3, sg4,
                   so0, so1, so2, so3, so4):
        bufs = [b0, b1, b2, b3, b4]
        g_sems = [sg0, sg1, sg2, sg3, sg4]
        o_sems = [so0, so1, so2, so3, so4]

        wid = lax.axis_index("s") * NC + lax.axis_index("c")
        base = wid * B_PER_W
        pltpu.sync_copy(x_hbm.at[pl.ds(base, B_PER_W)], idx_all)

        def gather_desc(i, p):
            idx = idx_all.at[pl.ds(i * SUB, SUB)]
            return pltpu.make_async_copy(table_hbm.at[idx], bufs[p],
                                         g_sems[p])

        def out_desc(i, p):
            dst = out_hbm.at[pl.ds(base + i * SUB, SUB)]
            return pltpu.make_async_copy(bufs[p], dst, o_sems[p])


        gather_desc(0, 0).start()
        for b in range(NRING):
            if b + 1 >= NRING:
                out_desc(b - NRING + 1, (b + 1) % NRING).wait()
            gather_desc(b + 1, (b + 1) % NRING).start()
            gather_desc(b, b).wait()
            out_desc(b, b).start()

        @pl.loop(1, N_GROUP - 1)
        def _group(g):
            i0 = g * NRING
            for b in range(NRING):
                i = i0 + b
                q = (b + 1) % NRING
                out_desc(i - NRING + 1, q).wait()
                gather_desc(i + 1, q).start()
                gather_desc(i, b).wait()
                out_desc(i, b).start()

        i0 = (N_GROUP - 1) * NRING
        for b in range(NRING):
            i = i0 + b
            q = (b + 1) % NRING
            out_desc(i - NRING + 1, q).wait()
            if b + 1 < NRING:
                gather_desc(i + 1, q).start()
            gather_desc(i, b).wait()
            out_desc(i, b).start()
        for b in range(1, NRING):
            out_desc(i0 + b, b).wait()

    return emb_kernel


_emb = _make_kernel()

_VB = 2048


def _tc_concat_body(g_ref, r_ref, o_ref):
    o_ref[:, 0:DIM] = g_ref[...].T
    o_ref[:, DIM:2 * DIM] = r_ref[...].T


_tc_concat = pl.pallas_call(
    _tc_concat_body,
    grid=(pl.cdiv(VOCAB, _VB),),
    in_specs=[
        pl.BlockSpec((DIM, _VB), lambda i: (0, i)),
        pl.BlockSpec((DIM, _VB), lambda i: (0, i)),
    ],
    out_specs=pl.BlockSpec((_VB, 2 * DIM), lambda i: (i, 0)),
    out_shape=jax.ShapeDtypeStruct((VOCAB, 2 * DIM), jnp.float32),
)


@jax.jit
def kernel(x, glove_table, rand_table):
    comb = _tc_concat(glove_table.T, rand_table.T)
    x_flat = x.reshape(B).astype(jnp.int32)
    out = _emb(x_flat, comb)
    return out.reshape(BATCH, NB_WORDS, 2 * DIM)


# speedup vs baseline: 11.1324x; 1.0826x over previous
import functools

import jax
import jax.numpy as jnp
from jax import lax
from jax.experimental import pallas as pl
from jax.experimental.pallas import tpu as pltpu
from jax.experimental.pallas import tpu_sc as plsc

VOCAB = 100000
DIM = 64
BATCH = 1024
NB_WORDS = 200
B = BATCH * NB_WORDS

_info = plsc.get_sparse_core_info()
NC, NS = _info.num_cores, _info.num_subcores
NW = NC * NS
B_PER_W = B // NW
SUB = 128
NRING = 5
N_SUB = B_PER_W // SUB
N_GROUP = N_SUB // NRING


def _make_kernel():
    mesh = plsc.VectorSubcoreMesh(core_axis_name="c", subcore_axis_name="s")

    ring_buf = pltpu.VMEM((SUB, 2 * DIM), jnp.float32)

    @functools.partial(
        pl.kernel,
        mesh=mesh,
        out_type=jax.ShapeDtypeStruct((B, 2 * DIM), jnp.float32),
        compiler_params=pltpu.CompilerParams(use_tc_tiling_on_sc=False),
        scratch_types=(
            [pltpu.VMEM((B_PER_W,), jnp.int32)]
            + [ring_buf] * NRING
            + [pltpu.SemaphoreType.DMA] * NRING
            + [pltpu.SemaphoreType.DMA] * NRING
        ),
    )
    def emb_kernel(x_hbm, table_hbm, out_hbm, idx_all,
                   b0, b1, b2, b3, b4,
                   sg0, sg1, sg2, sg3, sg4,
                   so0, so1, so2, so3, so4):
        bufs = [b0, b1, b2, b3, b4]
        g_sems = [sg0, sg1, sg2, sg3, sg4]
        o_sems = [so0, so1, so2, so3, so4]

        wid = lax.axis_index("s") * NC + lax.axis_index("c")
        base = wid * B_PER_W
        pltpu.sync_copy(x_hbm.at[pl.ds(base, B_PER_W)], idx_all)

        def gather_desc(i, p):
            idx = idx_all.at[pl.ds(i * SUB, SUB)]
            return pltpu.make_async_copy(table_hbm.at[idx], bufs[p],
                                         g_sems[p])

        def out_desc(i, p):
            dst = out_hbm.at[pl.ds(base + i * SUB, SUB)]
            return pltpu.make_async_copy(bufs[p], dst, o_sems[p])


        gather_desc(0, 0).start()
        for b in range(NRING):
            if b + 1 >= NRING:
                out_desc(b - NRING + 1, (b + 1) % NRING).wait()
            gather_desc(b + 1, (b + 1) % NRING).start()
            gather_desc(b, b).wait()
            out_desc(b, b).start()

        @pl.loop(1, N_GROUP - 1)
        def _group(g):
            i0 = g * NRING
            for b in range(NRING):
                i = i0 + b
                q = (b + 1) % NRING
                out_desc(i - NRING + 1, q).wait()
                gather_desc(i + 1, q).start()
                gather_desc(i, b).wait()
                out_desc(i, b).start()

        i0 = (N_GROUP - 1) * NRING
        for b in range(NRING):
            i = i0 + b
            q = (b + 1) % NRING
            out_desc(i - NRING + 1, q).wait()
            if b + 1 < NRING:
                gather_desc(i + 1, q).start()
            gather_desc(i, b).wait()
            out_desc(i, b).start()
        for b in range(1, NRING):
            out_desc(i0 + b, b).wait()

    return emb_kernel


_emb = _make_kernel()

_VB = 4096


def _tc_concat_body(g_ref, r_ref, o_ref):
    eye = (lax.broadcasted_iota(jnp.int32, (DIM, DIM), 0)
           == lax.broadcasted_iota(jnp.int32, (DIM, DIM), 1)
           ).astype(jnp.float32)
    dn = (((0,), (0,)), ((), ()))
    o_ref[:, 0:DIM] = lax.dot_general(g_ref[...], eye, dn,
                                      preferred_element_type=jnp.float32)
    o_ref[:, DIM:2 * DIM] = lax.dot_general(r_ref[...], eye, dn,
                                            preferred_element_type=jnp.float32)


_tc_concat = pl.pallas_call(
    _tc_concat_body,
    grid=(pl.cdiv(VOCAB, _VB),),
    in_specs=[
        pl.BlockSpec((DIM, _VB), lambda i: (0, i)),
        pl.BlockSpec((DIM, _VB), lambda i: (0, i)),
    ],
    out_specs=pl.BlockSpec((_VB, 2 * DIM), lambda i: (i, 0)),
    out_shape=jax.ShapeDtypeStruct((VOCAB, 2 * DIM), jnp.float32),
)


@jax.jit
def kernel(x, glove_table, rand_table):
    comb = _tc_concat(glove_table.T, rand_table.T)
    x_flat = x.reshape(B).astype(jnp.int32)
    out = _emb(x_flat, comb)
    return out.reshape(BATCH, NB_WORDS, 2 * DIM)


# speedup vs baseline: 11.5119x; 1.0341x over previous
import functools

import jax
import jax.numpy as jnp
from jax import lax
from jax.experimental import pallas as pl
from jax.experimental.pallas import tpu as pltpu
from jax.experimental.pallas import tpu_sc as plsc

VOCAB = 100000
DIM = 64
BATCH = 1024
NB_WORDS = 200
B = BATCH * NB_WORDS

_info = plsc.get_sparse_core_info()
NC, NS = _info.num_cores, _info.num_subcores
NW = NC * NS
B_PER_W = B // NW
SUB = 128
NRING = 5
N_SUB = B_PER_W // SUB
N_GROUP = N_SUB // NRING


def _make_kernel():
    mesh = plsc.VectorSubcoreMesh(core_axis_name="c", subcore_axis_name="s")

    ring_buf = pltpu.VMEM((SUB, 2 * DIM), jnp.float32)

    @functools.partial(
        pl.kernel,
        mesh=mesh,
        out_type=jax.ShapeDtypeStruct((B, 2 * DIM), jnp.float32),
        compiler_params=pltpu.CompilerParams(use_tc_tiling_on_sc=False),
        scratch_types=(
            [pltpu.VMEM((B_PER_W,), jnp.int32)]
            + [ring_buf] * NRING
            + [pltpu.SemaphoreType.DMA] * NRING
            + [pltpu.SemaphoreType.DMA] * NRING
        ),
    )
    def emb_kernel(x_hbm, table_hbm, out_hbm, idx_all,
                   b0, b1, b2, b3, b4,
                   sg0, sg1, sg2, sg3, sg4,
                   so0, so1, so2, so3, so4):
        bufs = [b0, b1, b2, b3, b4]
        g_sems = [sg0, sg1, sg2, sg3, sg4]
        o_sems = [so0, so1, so2, so3, so4]

        wid = lax.axis_index("s") * NC + lax.axis_index("c")
        base = wid * B_PER_W
        pltpu.sync_copy(x_hbm.at[pl.ds(base, B_PER_W)], idx_all)

        def gather_desc(i, p):
            idx = idx_all.at[pl.ds(i * SUB, SUB)]
            return pltpu.make_async_copy(table_hbm.at[idx], bufs[p],
                                         g_sems[p])

        def out_desc(i, p):
            dst = out_hbm.at[pl.ds(base + i * SUB, SUB)]
            return pltpu.make_async_copy(bufs[p], dst, o_sems[p])


        gather_desc(0, 0).start()
        for b in range(NRING):
            if b + 1 >= NRING:
                out_desc(b - NRING + 1, (b + 1) % NRING).wait()
            gather_desc(b + 1, (b + 1) % NRING).start()
            gather_desc(b, b).wait()
            out_desc(b, b).start()

        @pl.loop(1, N_GROUP - 1)
        def _group(g):
            i0 = g * NRING
            for b in range(NRING):
                i = i0 + b
                q = (b + 1) % NRING
                out_desc(i - NRING + 1, q).wait()
                gather_desc(i + 1, q).start()
                gather_desc(i, b).wait()
                out_desc(i, b).start()

        i0 = (N_GROUP - 1) * NRING
        for b in range(NRING):
            i = i0 + b
            q = (b + 1) % NRING
            out_desc(i - NRING + 1, q).wait()
            if b + 1 < NRING:
                gather_desc(i + 1, q).start()
            gather_desc(i, b).wait()
            out_desc(i, b).start()
        for b in range(1, NRING):
            out_desc(i0 + b, b).wait()

    return emb_kernel


_emb = _make_kernel()

_VB = 8192


def _tc_concat_body(g_ref, r_ref, o_ref):
    eye = (lax.broadcasted_iota(jnp.int32, (DIM, DIM), 0)
           == lax.broadcasted_iota(jnp.int32, (DIM, DIM), 1)
           ).astype(jnp.float32)
    dn = (((0,), (0,)), ((), ()))
    o_ref[:, 0:DIM] = lax.dot_general(g_ref[...], eye, dn,
                                      preferred_element_type=jnp.float32)
    o_ref[:, DIM:2 * DIM] = lax.dot_general(r_ref[...], eye, dn,
                                            preferred_element_type=jnp.float32)


_tc_concat = pl.pallas_call(
    _tc_concat_body,
    grid=(pl.cdiv(VOCAB, _VB),),
    in_specs=[
        pl.BlockSpec((DIM, _VB), lambda i: (0, i)),
        pl.BlockSpec((DIM, _VB), lambda i: (0, i)),
    ],
    out_specs=pl.BlockSpec((_VB, 2 * DIM), lambda i: (i, 0)),
    out_shape=jax.ShapeDtypeStruct((VOCAB, 2 * DIM), jnp.float32),
)


@jax.jit
def kernel(x, glove_table, rand_table):
    comb = _tc_concat(glove_table.T, rand_table.T)
    x_flat = x.reshape(B).astype(jnp.int32)
    out = _emb(x_flat, comb)
    return out.reshape(BATCH, NB_WORDS, 2 * DIM)


# speedup vs baseline: 11.5249x; 1.0011x over previous
import functools

import jax
import jax.numpy as jnp
from jax import lax
from jax.experimental import pallas as pl
from jax.experimental.pallas import tpu as pltpu
from jax.experimental.pallas import tpu_sc as plsc

VOCAB = 100000
DIM = 64
BATCH = 1024
NB_WORDS = 200
B = BATCH * NB_WORDS

_info = plsc.get_sparse_core_info()
NC, NS = _info.num_cores, _info.num_subcores
NW = NC * NS
B_PER_W = B // NW
SUB = 128
NRING = 5
N_SUB = B_PER_W // SUB
N_GROUP = N_SUB // NRING


def _make_kernel():
    mesh = plsc.VectorSubcoreMesh(core_axis_name="c", subcore_axis_name="s")

    ring_buf = pltpu.VMEM((SUB, 2 * DIM), jnp.float32)

    @functools.partial(
        pl.kernel,
        mesh=mesh,
        out_type=jax.ShapeDtypeStruct((B, 2 * DIM), jnp.float32),
        compiler_params=pltpu.CompilerParams(use_tc_tiling_on_sc=False),
        scratch_types=(
            [pltpu.VMEM((B_PER_W,), jnp.int32)]
            + [ring_buf] * NRING
            + [pltpu.SemaphoreType.DMA] * NRING
            + [pltpu.SemaphoreType.DMA] * NRING
        ),
    )
    def emb_kernel(x_hbm, table_hbm, out_hbm, idx_all,
                   b0, b1, b2, b3, b4,
                   sg0, sg1, sg2, sg3, sg4,
                   so0, so1, so2, so3, so4):
        bufs = [b0, b1, b2, b3, b4]
        g_sems = [sg0, sg1, sg2, sg3, sg4]
        o_sems = [so0, so1, so2, so3, so4]

        wid = lax.axis_index("s") * NC + lax.axis_index("c")
        base = wid * B_PER_W
        pltpu.sync_copy(x_hbm.at[pl.ds(base, B_PER_W)], idx_all)

        def gather_desc(i, p):
            idx = idx_all.at[pl.ds(i * SUB, SUB)]
            return pltpu.make_async_copy(table_hbm.at[idx], bufs[p],
                                         g_sems[p])

        def out_desc(i, p):
            dst = out_hbm.at[pl.ds(base + i * SUB, SUB)]
            return pltpu.make_async_copy(bufs[p], dst, o_sems[p])


        gather_desc(0, 0).start()
        for b in range(NRING):
            if b + 1 >= NRING:
                out_desc(b - NRING + 1, (b + 1) % NRING).wait()
            gather_desc(b + 1, (b + 1) % NRING).start()
            gather_desc(b, b).wait()
            out_desc(b, b).start()

        @pl.loop(1, N_GROUP - 1)
        def _group(g):
            i0 = g * NRING
            for b in range(NRING):
                i = i0 + b
                q = (b + 1) % NRING
                out_desc(i - NRING + 1, q).wait()
                gather_desc(i + 1, q).start()
                gather_desc(i, b).wait()
                out_desc(i, b).start()

        i0 = (N_GROUP - 1) * NRING
        for b in range(NRING):
            i = i0 + b
            q = (b + 1) % NRING
            out_desc(i - NRING + 1, q).wait()
            if b + 1 < NRING:
                gather_desc(i + 1, q).start()
            gather_desc(i, b).wait()
            out_desc(i, b).start()
        for b in range(1, NRING):
            out_desc(i0 + b, b).wait()

    return emb_kernel


_emb = _make_kernel()

_VB = 8192


def _tc_concat_body(g_ref, r_ref, o_ref):
    eye = (lax.broadcasted_iota(jnp.int32, (DIM, DIM), 0)
           == lax.broadcasted_iota(jnp.int32, (DIM, DIM), 1)
           ).astype(jnp.float32)
    dn = (((0,), (0,)), ((), ()))
    o_ref[:, 0:DIM] = g_ref[...].T
    o_ref[:, DIM:2 * DIM] = lax.dot_general(r_ref[...], eye, dn,
                                            preferred_element_type=jnp.float32)


_tc_concat = pl.pallas_call(
    _tc_concat_body,
    grid=(pl.cdiv(VOCAB, _VB),),
    in_specs=[
        pl.BlockSpec((DIM, _VB), lambda i: (0, i)),
        pl.BlockSpec((DIM, _VB), lambda i: (0, i)),
    ],
    out_specs=pl.BlockSpec((_VB, 2 * DIM), lambda i: (i, 0)),
    out_shape=jax.ShapeDtypeStruct((VOCAB, 2 * DIM), jnp.float32),
)


@jax.jit
def kernel(x, glove_table, rand_table):
    comb = _tc_concat(glove_table.T, rand_table.T)
    x_flat = x.reshape(B).astype(jnp.int32)
    out = _emb(x_flat, comb)
    return out.reshape(BATCH, NB_WORDS, 2 * DIM)
